# baseline (speedup 1.0000x reference)
import functools

import jax
import jax.numpy as jnp
from jax import lax
from jax.experimental import pallas as pl
from jax.experimental.pallas import tpu as pltpu
from jax.experimental.pallas import tpu_sc as plsc

D_MODEL = 768
SEQ_LEN = 2048
BATCH = 4

NUM_WORKERS = 32
T_PER_W = SEQ_LEN // NUM_WORKERS
SUB = 32
H = T_PER_W // SUB
NSUB = BATCH * H
VECS = D_MODEL // 16

_mesh = plsc.VectorSubcoreMesh(
    core_axis_name="c", subcore_axis_name="s", num_cores=2, num_subcores=16
)


@functools.partial(
    pl.kernel,
    out_type=jax.ShapeDtypeStruct((BATCH * SEQ_LEN, D_MODEL), jnp.float32),
    mesh=_mesh,
    scratch_types=[
        pltpu.VMEM((NSUB, SUB), jnp.int32),
        pltpu.VMEM((T_PER_W, D_MODEL), jnp.float32),
        pltpu.VMEM((SUB, D_MODEL), jnp.float32),
        pltpu.VMEM((SUB, D_MODEL), jnp.float32),
        pltpu.SemaphoreType.DMA,
        pltpu.SemaphoreType.DMA,
        pltpu.SemaphoreType.DMA,
        pltpu.SemaphoreType.DMA,
        pltpu.SemaphoreType.DMA,
    ],
)
def _embed_sc(idx_hbm, w_hbm, pos_hbm, out_hbm, idx_v, pos_v, buf0, buf1,
              pos_sem, g_sem0, g_sem1, st_sem0, st_sem1):
    wid = lax.axis_index("s") * 2 + lax.axis_index("c")
    t0 = wid * T_PER_W

    pltpu.sync_copy(idx_hbm.at[wid], idx_v)
    pos_desc = pltpu.async_copy(pos_hbm.at[pl.ds(t0, T_PER_W)], pos_v, pos_sem)

    bufs = (buf0, buf1)
    g_sems = (g_sem0, g_sem1)
    st_sems = (st_sem0, st_sem1)

    def out_slice(k):
        b, h = divmod(k, H)
        return out_hbm.at[pl.ds(b * SEQ_LEN + t0 + h * SUB, SUB)]

    g_descs = [None] * NSUB
    st_descs = [None] * NSUB
    g_descs[0] = pltpu.async_copy(w_hbm.at[idx_v.at[0]], buf0, g_sems[0])
    for k in range(NSUB):
        buf = bufs[k % 2]
        g_descs[k].wait()
        if k + 1 < NSUB:
            if k >= 1:
                st_descs[k - 1].wait()
            g_descs[k + 1] = pltpu.async_copy(
                w_hbm.at[idx_v.at[k + 1]], bufs[(k + 1) % 2],
                g_sems[(k + 1) % 2])
        if k == 0:
            pos_desc.wait()
        h = k % H
        @pl.loop(0, SUB)
        def _add_row(r):
            for v in range(VECS):
                sl = pl.ds(v * 16, 16)
                plsc.addupdate(buf.at[r, sl], pos_v[h * SUB + r, sl])
        st_descs[k] = pltpu.async_copy(buf, out_slice(k), st_sems[k % 2])
    st_descs[NSUB - 2].wait()
    st_descs[NSUB - 1].wait()


def kernel(x, W, pos_embd):
    idx = (x.astype(jnp.int32)
           .reshape(BATCH, NUM_WORKERS, H, SUB)
           .transpose(1, 0, 2, 3)
           .reshape(NUM_WORKERS, NSUB, SUB))
    out = _embed_sc(idx, W, pos_embd)
    return out.reshape(BATCH, SEQ_LEN, D_MODEL)


# speedup vs baseline: 1.1342x; 1.1342x over previous
import functools

import jax
import jax.numpy as jnp
from jax import lax
from jax.experimental import pallas as pl
from jax.experimental.pallas import tpu as pltpu
from jax.experimental.pallas import tpu_sc as plsc

D_MODEL = 768
SEQ_LEN = 2048
BATCH = 4

NUM_WORKERS = 32
T_PER_W = SEQ_LEN // NUM_WORKERS
SUB = 32
H = T_PER_W // SUB
NSUB = BATCH * H
VECS = D_MODEL // 16

_mesh = plsc.VectorSubcoreMesh(
    core_axis_name="c", subcore_axis_name="s", num_cores=2, num_subcores=16
)


@functools.partial(
    pl.kernel,
    out_type=jax.ShapeDtypeStruct((BATCH * SEQ_LEN, D_MODEL), jnp.float32),
    mesh=_mesh,
    scratch_types=[
        pltpu.VMEM((BATCH, T_PER_W), jnp.int32),
        pltpu.VMEM((T_PER_W, D_MODEL), jnp.float32),
        pltpu.VMEM((SUB, D_MODEL), jnp.float32),
        pltpu.VMEM((SUB, D_MODEL), jnp.float32),
        pltpu.SemaphoreType.DMA,
        pltpu.SemaphoreType.DMA,
        pltpu.SemaphoreType.DMA,
        pltpu.SemaphoreType.DMA,
        pltpu.SemaphoreType.DMA,
    ],
)
def _embed_sc(idx_hbm, w_hbm, pos_hbm, out_hbm, idx_v, pos_v, buf0, buf1,
              pos_sem, g_sem0, g_sem1, st_sem0, st_sem1):
    wid = lax.axis_index("s") * 2 + lax.axis_index("c")
    t0 = wid * T_PER_W

    for b in range(BATCH):
        pltpu.sync_copy(idx_hbm.at[pl.ds(b * SEQ_LEN + t0, T_PER_W)],
                        idx_v.at[b])
    pos_desc = pltpu.async_copy(pos_hbm.at[pl.ds(t0, T_PER_W)], pos_v, pos_sem)

    bufs = (buf0, buf1)
    g_sems = (g_sem0, g_sem1)
    st_sems = (st_sem0, st_sem1)

    def gather(k):
        b, h = divmod(k, H)
        return pltpu.async_copy(
            w_hbm.at[idx_v.at[b, pl.ds(h * SUB, SUB)]], bufs[k % 2],
            g_sems[k % 2])

    def out_slice(k):
        b, h = divmod(k, H)
        return out_hbm.at[pl.ds(b * SEQ_LEN + t0 + h * SUB, SUB)]

    g_descs = [None] * NSUB
    st_descs = [None] * NSUB
    g_descs[0] = gather(0)
    for k in range(NSUB):
        buf = bufs[k % 2]
        g_descs[k].wait()
        if k + 1 < NSUB:
            if k >= 1:
                st_descs[k - 1].wait()
            g_descs[k + 1] = gather(k + 1)
        if k == 0:
            pos_desc.wait()
        h = k % H

        @plsc.parallel_loop(0, SUB, unroll=2)
        def _add_row(r):
            for v in range(VECS):
                sl = pl.ds(v * 16, 16)
                plsc.addupdate(buf.at[r, sl], pos_v[h * SUB + r, sl])

        st_descs[k] = pltpu.async_copy(buf, out_slice(k), st_sems[k % 2])
    st_descs[NSUB - 2].wait()
    st_descs[NSUB - 1].wait()


def kernel(x, W, pos_embd):
    idx = x.astype(jnp.int32).reshape(BATCH * SEQ_LEN)
    out = _embed_sc(idx, W, pos_embd)
    return out.reshape(BATCH, SEQ_LEN, D_MODEL)
